# baseline (speedup 1.0000x reference)
import functools
import jax
import jax.numpy as jnp
from jax import lax
from jax.experimental import pallas as pl
from jax.experimental.pallas import tpu as pltpu
from jax.experimental.pallas import tpu_sc as plsc

V, D, B = 1000000, 64, 16384
NC, NS = 2, 16
NW = NC * NS
B_PER_W = B // NW
CH = 64
NCH = B_PER_W // CH
L = 16

_mesh = plsc.VectorSubcoreMesh(core_axis_name="c", subcore_axis_name="s")


@functools.partial(
    pl.kernel,
    mesh=_mesh,
    out_type=jax.ShapeDtypeStruct((B, D), jnp.float32),
    scratch_types=[
        pltpu.VMEM((NCH, CH), jnp.int32),
        pltpu.VMEM((B_PER_W,), jnp.int32),
        pltpu.VMEM((CH, 8, 2 * D), jnp.float32),
        pltpu.VMEM((CH, D), jnp.float32),
        pltpu.SemaphoreType.DMA,
    ],
)
def _gather_sc(table3, bidx_hbm, sidx_hbm, out_hbm,
               bidx_v, sidx_v, rows_v, out_v, sem):
    wid = lax.axis_index("s") * NC + lax.axis_index("c")
    base = wid * B_PER_W
    pltpu.sync_copy(bidx_hbm.at[wid], bidx_v)
    pltpu.sync_copy(sidx_hbm.at[wid], sidx_v)
    for j in range(NCH):
        pltpu.async_copy(table3.at[bidx_v.at[j]], rows_v, sem).wait()
        for g in range(CH // L):
            s_vec = sidx_v[pl.ds(j * CH + g * L, L)]
            for l in range(L):
                i = g * L + l
                s = s_vec[l]
                r = s >> 1
                h = (s & 1) * D
                for c in range(D // L):
                    out_v[i, pl.ds(c * L, L)] = rows_v[i, r, pl.ds(h + c * L, L)]
        pltpu.sync_copy(out_v, out_hbm.at[pl.ds(base + j * CH, CH)])


def kernel(input, indices):
    idx = indices.astype(jnp.int32)
    table3 = input.reshape(V // 16, 8, 2 * D)
    bidx = (idx >> 4).reshape(NW, NCH, CH)
    sidx = (idx & 15).reshape(NW, B_PER_W)
    return _gather_sc(table3, bidx, sidx)
